# baseline (speedup 1.0000x reference)
import functools

import jax
import jax.numpy as jnp
from jax import lax
from jax.experimental import pallas as pl
from jax.experimental.pallas import tpu as pltpu
from jax.experimental.pallas import tpu_sc as plsc

_C = 10
_E = 64
_A = 32


_H = 507904


def _relayout(tTu, tTi):
    BL = 4096
    nb = _H // BL

    def one(a, b):
        i0 = lax.broadcasted_iota(jnp.int32, (128, 128), 0)
        i1 = lax.broadcasted_iota(jnp.int32, (128, 128), 1)
        eye = (i0 == i1).astype(jnp.float32)
        ab = jnp.concatenate([a, b], axis=0)
        dn = (((0,), (0,)), ((), ()))
        hi = ab.astype(jnp.bfloat16).astype(jnp.float32)
        lo = ab - hi
        return (lax.dot_general(hi, eye, dn,
                                preferred_element_type=jnp.float32)
                + lax.dot_general(lo, eye, dn,
                                  preferred_element_type=jnp.float32))

    def body(au_ref, bu_ref, ai_ref, bi_ref, ou_ref, oi_ref):
        ou_ref[...] = one(au_ref[...], bu_ref[...])
        oi_ref[...] = one(ai_ref[...], bi_ref[...])

    last = (tTu.shape[1] + BL - 1) // BL - 1
    aspec = pl.BlockSpec((_E, BL), lambda i: (0, i))
    bspec = pl.BlockSpec((_E, BL),
                         lambda i, last=last: (0, jnp.minimum(i + nb, last)))
    return pl.pallas_call(
        body,
        grid=(nb,),
        in_specs=[aspec, bspec, aspec, bspec],
        out_specs=[pl.BlockSpec((BL, 128), lambda i: (i, 0))] * 2,
        out_shape=[jax.ShapeDtypeStruct((_H, 128), jnp.float32)] * 2,
    )(tTu, tTu, tTi, tTi)


def _tc_attend(ids, ct, W, b_row, v, B):
    bm = 4096
    nb = B // bm

    def body(ids_ref, ct_ref, w_ref, b_ref, v_ref, out_ref):
        ct_full = ct_ref[...]
        q = jnp.tanh(
            jnp.dot(ct_full, w_ref[...],
                    preferred_element_type=jnp.float32) + b_ref[...])
        logit = jnp.dot(q, v_ref[...],
                        preferred_element_type=jnp.float32)
        e = jnp.exp(logit - jnp.max(logit, axis=0, keepdims=True))
        idv = ids_ref[...]
        den = jnp.zeros((bm, 1), jnp.float32)
        ca_num = jnp.zeros((bm, _E), jnp.float32)
        for g in range(_C):
            n_g = jnp.sum((idv == g).astype(jnp.float32), axis=1,
                          keepdims=True)
            e_g = lax.slice(e, (g, 0), (g + 1, 1))
            w_g = n_g * e_g
            den = den + w_g
            ca_num = ca_num + w_g * lax.slice(ct_full, (g, 0), (g + 1, _E))
        out_ref[...] = ca_num / den

    return pl.pallas_call(
        body,
        grid=(nb,),
        in_specs=[
            pl.BlockSpec((bm, _C), lambda i: (i, 0)),
            pl.BlockSpec((_C, _E), lambda i: (0, 0)),
            pl.BlockSpec((_E, _A), lambda i: (0, 0)),
            pl.BlockSpec((1, _A), lambda i: (0, 0)),
            pl.BlockSpec((_A, 1), lambda i: (0, 0)),
        ],
        out_specs=pl.BlockSpec((bm, _E), lambda i: (i, 0)),
        out_shape=jax.ShapeDtypeStruct((B, _E), jnp.float32),
    )(ids, ct, W, b_row, v)


def _sc_gather_score(user_ids, item_ids, u2, i2, ca):
    B = user_ids.shape[0]
    H = u2.shape[0]
    info = plsc.get_sparse_core_info()
    nw = info.num_cores * info.num_subcores
    bpw = B // nw
    nh = 2
    hb = bpw // nh
    ch = 128
    nch = hb // ch

    mesh = plsc.VectorSubcoreMesh(core_axis_name="c", subcore_axis_name="s")

    @functools.partial(
        pl.kernel,
        mesh=mesh,
        compiler_params=pltpu.CompilerParams(use_tc_tiling_on_sc=False),
        out_type=jax.ShapeDtypeStruct((B,), jnp.float32),
        scratch_types=[
            pltpu.VMEM((hb,), jnp.int32),
            pltpu.VMEM((hb,), jnp.int32),
            pltpu.VMEM((hb,), jnp.int32),
            pltpu.VMEM((hb,), jnp.int32),
            pltpu.VMEM((hb, 128), jnp.float32),
            pltpu.VMEM((hb, 128), jnp.float32),
            pltpu.VMEM((hb, _E), jnp.float32),
            pltpu.VMEM((hb,), jnp.float32),
            pltpu.SemaphoreType.DMA,
        ],
    )
    def body(uid_hbm, iid_hbm, u2_hbm, i2_hbm, ca_hbm, out,
             ublk, iblk, us, iscal, urows, irows, cav, svec, sem):
        wid = lax.axis_index("s") * info.num_cores + lax.axis_index("c")
        for half in range(nh):
            base = wid * bpw + half * hb
            pltpu.sync_copy(uid_hbm.at[pl.ds(base, hb)], ublk)
            pltpu.sync_copy(iid_hbm.at[pl.ds(base, hb)], iblk)
            pltpu.sync_copy(uid_hbm.at[pl.ds(base, hb)], us)
            pltpu.sync_copy(iid_hbm.at[pl.ds(base, hb)], iscal)

            def mkblk(c, _):
                sl = pl.ds(c * 16, 16)
                iu = ublk[sl]
                ii = iblk[sl]
                ublk[sl] = jnp.where(iu >= H, iu - H, iu)
                iblk[sl] = jnp.where(ii >= H, ii - H, ii)
                return _
            lax.fori_loop(0, hb // 16, mkblk, 0)

            descs = [pltpu.async_copy(ca_hbm.at[pl.ds(base, hb)], cav, sem)]
            for c in range(nch):
                sl = pl.ds(c * ch, ch)
                descs.append(pltpu.async_copy(
                    u2_hbm.at[ublk.at[sl]], urows.at[sl], sem))
                descs.append(pltpu.async_copy(
                    i2_hbm.at[iblk.at[sl]], irows.at[sl], sem))
            for d in descs:
                d.wait()

            lane = lax.broadcasted_iota(jnp.int32, (16,), 0)
            perms = [jnp.reshape(lane ^ m, (16, 1)) for m in (8, 4, 2, 1)]
            dnums = lax.GatherDimensionNumbers(
                offset_dims=(), collapsed_slice_dims=(0,),
                start_index_map=(0,))

            def lanesum(x):
                for p in perms:
                    x = x + lax.gather(
                        x, p, dnums, (1,),
                        mode=lax.GatherScatterMode.PROMISE_IN_BOUNDS)
                return x

            def red(g, _):
                acc16 = jnp.zeros((16,), jnp.float32)
                gs16 = pl.ds(g * 16, 16)
                offuv = jnp.where(us[gs16] >= H, _E, 0)
                offiv = jnp.where(iscal[gs16] >= H, _E, 0)
                for l in range(16):
                    j = g * 16 + l
                    offu = offuv[l]
                    offi = offiv[l]
                    acc = jnp.zeros((16,), jnp.float32)
                    for k in range(_E // 16):
                        uv = urows[j, pl.ds(offu + k * 16, 16)]
                        iv = irows[j, pl.ds(offi + k * 16, 16)]
                        cv = cav[j, pl.ds(k * 16, 16)]
                        acc = acc + uv * (iv + cv)
                    acc16 = jnp.where(lane == l, lanesum(acc), acc16)
                svec[pl.ds(g * 16, 16)] = acc16
                return _
            lax.fori_loop(0, hb // 16, red, 0)
            pltpu.sync_copy(svec, out.at[pl.ds(base, hb)])

    return body(user_ids, item_ids, u2, i2, ca)


def kernel(user_ids, item_ids, component_ids, user_table, item_table,
           component_table, W, b, v):
    u2, i2 = _relayout(user_table.T, item_table.T)
    ca = _tc_attend(component_ids, component_table, W, b.reshape(1, _A), v,
                    user_ids.shape[0])
    return _sc_gather_score(user_ids, item_ids, u2, i2, ca)


# speedup vs baseline: 1.0635x; 1.0635x over previous
import functools

import jax
import jax.numpy as jnp
from jax import lax
from jax.experimental import pallas as pl
from jax.experimental.pallas import tpu as pltpu
from jax.experimental.pallas import tpu_sc as plsc

_C = 10
_E = 64
_A = 32


_H = 507904


def _relayout(tTu, tTi):
    BL = 8192
    nb = _H // BL

    def one(a, b):
        i0 = lax.broadcasted_iota(jnp.int32, (128, 128), 0)
        i1 = lax.broadcasted_iota(jnp.int32, (128, 128), 1)
        eye = (i0 == i1).astype(jnp.float32)
        ab = jnp.concatenate([a, b], axis=0)
        dn = (((0,), (0,)), ((), ()))
        hi = ab.astype(jnp.bfloat16).astype(jnp.float32)
        lo = ab - hi
        return (lax.dot_general(hi, eye, dn,
                                preferred_element_type=jnp.float32)
                + lax.dot_general(lo, eye, dn,
                                  preferred_element_type=jnp.float32))

    def body(au_ref, bu_ref, ai_ref, bi_ref, ou_ref, oi_ref):
        ou_ref[...] = one(au_ref[...], bu_ref[...])
        oi_ref[...] = one(ai_ref[...], bi_ref[...])

    last = (tTu.shape[1] + BL - 1) // BL - 1
    aspec = pl.BlockSpec((_E, BL), lambda i: (0, i))
    bspec = pl.BlockSpec((_E, BL),
                         lambda i, last=last: (0, jnp.minimum(i + nb, last)))
    return pl.pallas_call(
        body,
        grid=(nb,),
        in_specs=[aspec, bspec, aspec, bspec],
        out_specs=[pl.BlockSpec((BL, 128), lambda i: (i, 0))] * 2,
        out_shape=[jax.ShapeDtypeStruct((_H, 128), jnp.float32)] * 2,
    )(tTu, tTu, tTi, tTi)


def _tc_attend(ids, ct, W, b_row, v, B):
    bm = 4096
    nb = B // bm

    def body(ids_ref, ct_ref, w_ref, b_ref, v_ref, out_ref):
        ct_full = ct_ref[...]
        q = jnp.tanh(
            jnp.dot(ct_full, w_ref[...],
                    preferred_element_type=jnp.float32) + b_ref[...])
        logit = jnp.dot(q, v_ref[...],
                        preferred_element_type=jnp.float32)
        e = jnp.exp(logit - jnp.max(logit, axis=0, keepdims=True))
        idv = ids_ref[...]
        den = jnp.zeros((bm, 1), jnp.float32)
        ca_num = jnp.zeros((bm, _E), jnp.float32)
        for g in range(_C):
            n_g = jnp.sum((idv == g).astype(jnp.float32), axis=1,
                          keepdims=True)
            e_g = lax.slice(e, (g, 0), (g + 1, 1))
            w_g = n_g * e_g
            den = den + w_g
            ca_num = ca_num + w_g * lax.slice(ct_full, (g, 0), (g + 1, _E))
        out_ref[...] = ca_num / den

    return pl.pallas_call(
        body,
        grid=(nb,),
        in_specs=[
            pl.BlockSpec((bm, _C), lambda i: (i, 0)),
            pl.BlockSpec((_C, _E), lambda i: (0, 0)),
            pl.BlockSpec((_E, _A), lambda i: (0, 0)),
            pl.BlockSpec((1, _A), lambda i: (0, 0)),
            pl.BlockSpec((_A, 1), lambda i: (0, 0)),
        ],
        out_specs=pl.BlockSpec((bm, _E), lambda i: (i, 0)),
        out_shape=jax.ShapeDtypeStruct((B, _E), jnp.float32),
    )(ids, ct, W, b_row, v)


def _sc_gather_score(user_ids, item_ids, u2, i2, ca):
    B = user_ids.shape[0]
    H = u2.shape[0]
    info = plsc.get_sparse_core_info()
    nw = info.num_cores * info.num_subcores
    bpw = B // nw
    nh = 2
    hb = bpw // nh
    ch = 128
    nch = hb // ch

    mesh = plsc.VectorSubcoreMesh(core_axis_name="c", subcore_axis_name="s")

    @functools.partial(
        pl.kernel,
        mesh=mesh,
        compiler_params=pltpu.CompilerParams(use_tc_tiling_on_sc=False),
        out_type=jax.ShapeDtypeStruct((B,), jnp.float32),
        scratch_types=[
            pltpu.VMEM((hb,), jnp.int32),
            pltpu.VMEM((hb,), jnp.int32),
            pltpu.VMEM((hb,), jnp.int32),
            pltpu.VMEM((hb,), jnp.int32),
            pltpu.VMEM((hb, 128), jnp.float32),
            pltpu.VMEM((hb, 128), jnp.float32),
            pltpu.VMEM((hb, _E), jnp.float32),
            pltpu.VMEM((hb,), jnp.float32),
            pltpu.SemaphoreType.DMA,
        ],
    )
    def body(uid_hbm, iid_hbm, u2_hbm, i2_hbm, ca_hbm, out,
             ublk, iblk, us, iscal, urows, irows, cav, svec, sem):
        wid = lax.axis_index("s") * info.num_cores + lax.axis_index("c")
        for half in range(nh):
            base = wid * bpw + half * hb
            pltpu.sync_copy(uid_hbm.at[pl.ds(base, hb)], ublk)
            pltpu.sync_copy(iid_hbm.at[pl.ds(base, hb)], iblk)
            pltpu.sync_copy(uid_hbm.at[pl.ds(base, hb)], us)
            pltpu.sync_copy(iid_hbm.at[pl.ds(base, hb)], iscal)

            def mkblk(c, _):
                sl = pl.ds(c * 16, 16)
                iu = ublk[sl]
                ii = iblk[sl]
                ublk[sl] = jnp.where(iu >= H, iu - H, iu)
                iblk[sl] = jnp.where(ii >= H, ii - H, ii)
                return _
            lax.fori_loop(0, hb // 16, mkblk, 0)

            descs = [pltpu.async_copy(ca_hbm.at[pl.ds(base, hb)], cav, sem)]
            for c in range(nch):
                sl = pl.ds(c * ch, ch)
                descs.append(pltpu.async_copy(
                    u2_hbm.at[ublk.at[sl]], urows.at[sl], sem))
                descs.append(pltpu.async_copy(
                    i2_hbm.at[iblk.at[sl]], irows.at[sl], sem))
            for d in descs:
                d.wait()

            lane = lax.broadcasted_iota(jnp.int32, (16,), 0)
            perms = [jnp.reshape(lane ^ m, (16, 1)) for m in (8, 4, 2, 1)]
            dnums = lax.GatherDimensionNumbers(
                offset_dims=(), collapsed_slice_dims=(0,),
                start_index_map=(0,))

            def lanesum(x):
                for p in perms:
                    x = x + lax.gather(
                        x, p, dnums, (1,),
                        mode=lax.GatherScatterMode.PROMISE_IN_BOUNDS)
                return x

            def red(g, _):
                acc16 = jnp.zeros((16,), jnp.float32)
                gs16 = pl.ds(g * 16, 16)
                offuv = jnp.where(us[gs16] >= H, _E, 0)
                offiv = jnp.where(iscal[gs16] >= H, _E, 0)
                for l in range(16):
                    j = g * 16 + l
                    offu = offuv[l]
                    offi = offiv[l]
                    acc = jnp.zeros((16,), jnp.float32)
                    for k in range(_E // 16):
                        uv = urows[j, pl.ds(offu + k * 16, 16)]
                        iv = irows[j, pl.ds(offi + k * 16, 16)]
                        cv = cav[j, pl.ds(k * 16, 16)]
                        acc = acc + uv * (iv + cv)
                    acc16 = jnp.where(lane == l, lanesum(acc), acc16)
                svec[pl.ds(g * 16, 16)] = acc16
                return _
            lax.fori_loop(0, hb // 16, red, 0)
            pltpu.sync_copy(svec, out.at[pl.ds(base, hb)])

    return body(user_ids, item_ids, u2, i2, ca)


def kernel(user_ids, item_ids, component_ids, user_table, item_table,
           component_table, W, b, v):
    u2, i2 = _relayout(user_table.T, item_table.T)
    ca = _tc_attend(component_ids, component_table, W, b.reshape(1, _A), v,
                    user_ids.shape[0])
    return _sc_gather_score(user_ids, item_ids, u2, i2, ca)
